# baseline (speedup 1.0000x reference)
import functools

import jax
import jax.numpy as jnp
from jax import lax
from jax.experimental import pallas as pl
from jax.experimental.pallas import tpu as pltpu
from jax.experimental.pallas import tpu_sc as plsc

N = 16777216
NBIN = 256
INV_DH = float(NBIN - 1)

NC = 2
NS = 16
NW = NC * NS
LANES = 16

PER_W = N // NW
CH = 32768
NCH = PER_W // CH
HALF = CH // 2
UNROLL = 16

STRIDE = 273
HIST_WORDS = LANES * STRIDE + LANES
OUTW = 272

CNT_SHIFT = 20
FRAC_BITS = 9
FRAC_SCALE = float(1 << FRAC_BITS)
PACK_C = float(1 << CNT_SHIFT) + 0.5
FRAC_MASK = (1 << CNT_SHIFT) - 1
INV_FRAC_SCALE = 1.0 / FRAC_SCALE


@functools.cache
def _build_hist_kernel():
    mesh = plsc.VectorSubcoreMesh(core_axis_name="c", subcore_axis_name="s")
    return pl.kernel(
        _hist_body,
        out_type=jax.ShapeDtypeStruct((NW, OUTW), jnp.float32),
        mesh=mesh,
        compiler_params=pltpu.CompilerParams(needs_layout_passes=False),
        scratch_types=[
            pltpu.VMEM((CH,), jnp.float32),
            pltpu.VMEM((CH,), jnp.float32),
            pltpu.VMEM((HIST_WORDS,), jnp.int32),
            pltpu.VMEM((HIST_WORDS,), jnp.float32),
            pltpu.VMEM((HIST_WORDS,), jnp.float32),
            pltpu.VMEM((OUTW + LANES,), jnp.float32),
            pltpu.VMEM((OUTW,), jnp.float32),
            pltpu.SemaphoreType.DMA,
            pltpu.SemaphoreType.DMA,
        ],
    )


def _hist_body(
    img_hbm, out_hbm, buf0, buf1, phist, nacc, facc, frow, outrow, sem0, sem1
):
    wid = lax.axis_index("s") * NC + lax.axis_index("c")
    base = wid * PER_W

    zero = jnp.zeros((LANES,), jnp.float32)
    izero = jnp.zeros((LANES,), jnp.int32)

    def zbody(i, carry):
        phist[pl.ds(i * LANES, LANES)] = izero
        nacc[pl.ds(i * LANES, LANES)] = zero
        facc[pl.ds(i * LANES, LANES)] = zero
        return carry

    lax.fori_loop(0, HIST_WORDS // LANES, zbody, 0)

    lane_base = lax.iota(jnp.int32, LANES) * STRIDE

    def scan(bref):
        @plsc.parallel_loop(0, CH, LANES, unroll=UNROLL)
        def _(i):
            x = bref[pl.ds(i, LANES)]
            u = x * INV_DH
            idx = u.astype(jnp.int32)
            d = u - idx.astype(jnp.float32)
            pv = (d * FRAC_SCALE + PACK_C).astype(jnp.int32)
            plsc.addupdate_scatter(phist, [lane_base + idx], pv)

    def flush():
        @plsc.parallel_loop(0, HIST_WORDS, LANES, unroll=4)
        def _(g):
            s = pl.ds(g, LANES)
            p = phist[s]
            c = lax.shift_right_logical(p, CNT_SHIFT)
            fr = p & FRAC_MASK
            nacc[s] = nacc[s] + c.astype(jnp.float32)
            facc[s] = facc[s] + fr.astype(jnp.float32) * INV_FRAC_SCALE
            phist[s] = izero

    def process(bref):
        scan(bref)
        flush()

    def copy(c, bref, sem):
        return pltpu.make_async_copy(
            img_hbm.at[pl.ds(base + c * CH, CH)], bref, sem
        )

    copy(0, buf0, sem0).start()

    def chunk_pair(p, carry):
        c0 = 2 * p
        copy(c0, buf0, sem0).wait()
        copy(c0 + 1, buf1, sem1).start()
        process(buf0)
        copy(c0 + 1, buf1, sem1).wait()

        @pl.when(p < NCH // 2 - 1)
        def _():
            copy(c0 + 2, buf0, sem0).start()

        process(buf1)
        return carry

    lax.fori_loop(0, NCH // 2, chunk_pair, 0)

    frow[pl.ds(0, LANES)] = zero
    frow[pl.ds(OUTW, LANES)] = zero
    for v in range(OUTW // LANES):
        nrow = nacc[pl.ds(v * LANES, LANES)]
        fr = facc[pl.ds(v * LANES, LANES)]
        for l in range(1, LANES):
            off = l * STRIDE + v * LANES
            nrow = nrow + nacc[pl.ds(off, LANES)]
            fr = fr + facc[pl.ds(off, LANES)]
        outrow[pl.ds(v * LANES, LANES)] = nrow - fr
        frow[pl.ds(v * LANES + 1, LANES)] = fr
    for v in range(OUTW // LANES):
        s = pl.ds(v * LANES, LANES)
        outrow[s] = outrow[s] + frow[s]

    pltpu.sync_copy(outrow, out_hbm.at[wid])


def kernel(img):
    img = img.reshape(-1)
    parts = _build_hist_kernel()(img)
    return jnp.sum(parts[:, :NBIN], axis=0)


# speedup vs baseline: 1.0780x; 1.0780x over previous
import functools

import jax
import jax.numpy as jnp
from jax import lax
from jax.experimental import pallas as pl
from jax.experimental.pallas import tpu as pltpu
from jax.experimental.pallas import tpu_sc as plsc

N = 16777216
NBIN = 256
INV_DH = float(NBIN - 1)

NC = 2
NS = 16
NW = NC * NS
LANES = 16

PER_W = N // NW
CH = 32768
NCH = PER_W // CH
HALF = CH // 2
UNROLL = 8

STRIDE = 273
HIST_WORDS = LANES * STRIDE + LANES
OUTW = 272

CNT_SHIFT = 20
FRAC_BITS = 9
FRAC_SCALE = float(1 << FRAC_BITS)
PACK_C = float(1 << CNT_SHIFT) + 0.5
FRAC_MASK = (1 << CNT_SHIFT) - 1
INV_FRAC_SCALE = 1.0 / FRAC_SCALE


@functools.cache
def _build_hist_kernel():
    mesh = plsc.VectorSubcoreMesh(core_axis_name="c", subcore_axis_name="s")
    return pl.kernel(
        _hist_body,
        out_type=jax.ShapeDtypeStruct((NW, OUTW), jnp.float32),
        mesh=mesh,
        compiler_params=pltpu.CompilerParams(needs_layout_passes=False),
        scratch_types=[
            pltpu.VMEM((CH,), jnp.float32),
            pltpu.VMEM((CH,), jnp.float32),
            pltpu.VMEM((HIST_WORDS,), jnp.int32),
            pltpu.VMEM((HIST_WORDS,), jnp.float32),
            pltpu.VMEM((HIST_WORDS,), jnp.float32),
            pltpu.VMEM((OUTW + LANES,), jnp.float32),
            pltpu.VMEM((OUTW,), jnp.float32),
            pltpu.SemaphoreType.DMA,
            pltpu.SemaphoreType.DMA,
        ],
    )


def _hist_body(
    img_hbm, out_hbm, buf0, buf1, phist, nacc, facc, frow, outrow, sem0, sem1
):
    wid = lax.axis_index("s") * NC + lax.axis_index("c")
    base = wid * PER_W

    zero = jnp.zeros((LANES,), jnp.float32)
    izero = jnp.zeros((LANES,), jnp.int32)

    def zbody(i, carry):
        phist[pl.ds(i * LANES, LANES)] = izero
        nacc[pl.ds(i * LANES, LANES)] = zero
        facc[pl.ds(i * LANES, LANES)] = zero
        return carry

    lax.fori_loop(0, HIST_WORDS // LANES, zbody, 0)

    lane_base = lax.iota(jnp.int32, LANES) * STRIDE

    def scan(bref):
        @plsc.parallel_loop(0, CH, LANES, unroll=UNROLL)
        def _(i):
            x = bref[pl.ds(i, LANES)]
            u = x * INV_DH
            idx = u.astype(jnp.int32)
            d = u - idx.astype(jnp.float32)
            pv = (d * FRAC_SCALE + PACK_C).astype(jnp.int32)
            plsc.addupdate_scatter(phist, [lane_base + idx], pv)

    def flush():
        @plsc.parallel_loop(0, HIST_WORDS, LANES, unroll=4)
        def _(g):
            s = pl.ds(g, LANES)
            p = phist[s]
            c = lax.shift_right_logical(p, CNT_SHIFT)
            fr = p & FRAC_MASK
            nacc[s] = nacc[s] + c.astype(jnp.float32)
            facc[s] = facc[s] + fr.astype(jnp.float32) * INV_FRAC_SCALE
            phist[s] = izero

    def process(bref):
        scan(bref)
        flush()

    def copy(c, bref, sem):
        return pltpu.make_async_copy(
            img_hbm.at[pl.ds(base + c * CH, CH)], bref, sem
        )

    copy(0, buf0, sem0).start()

    def chunk_pair(p, carry):
        c0 = 2 * p
        copy(c0, buf0, sem0).wait()
        copy(c0 + 1, buf1, sem1).start()
        process(buf0)
        copy(c0 + 1, buf1, sem1).wait()

        @pl.when(p < NCH // 2 - 1)
        def _():
            copy(c0 + 2, buf0, sem0).start()

        process(buf1)
        return carry

    lax.fori_loop(0, NCH // 2, chunk_pair, 0)

    frow[pl.ds(0, LANES)] = zero
    frow[pl.ds(OUTW, LANES)] = zero
    for v in range(OUTW // LANES):
        nrow = nacc[pl.ds(v * LANES, LANES)]
        fr = facc[pl.ds(v * LANES, LANES)]
        for l in range(1, LANES):
            off = l * STRIDE + v * LANES
            nrow = nrow + nacc[pl.ds(off, LANES)]
            fr = fr + facc[pl.ds(off, LANES)]
        outrow[pl.ds(v * LANES, LANES)] = nrow - fr
        frow[pl.ds(v * LANES + 1, LANES)] = fr
    for v in range(OUTW // LANES):
        s = pl.ds(v * LANES, LANES)
        outrow[s] = outrow[s] + frow[s]

    pltpu.sync_copy(outrow, out_hbm.at[wid])


def kernel(img):
    img = img.reshape(-1)
    parts = _build_hist_kernel()(img)
    return jnp.sum(parts[:, :NBIN], axis=0)


# speedup vs baseline: 1.5124x; 1.4031x over previous
import functools

import jax
import jax.numpy as jnp
from jax import lax
from jax.experimental import pallas as pl
from jax.experimental.pallas import tpu as pltpu
from jax.experimental.pallas import tpu_sc as plsc

N = 16777216
NBIN = 256
INV_DH = float(NBIN - 1)

NC = 2
NS = 16
NW = NC * NS
LANES = 16

PER_W = N // NW
CH = 32768
NCH = PER_W // CH
HALF = CH // 2
UNROLL = 8

STRIDE = 273
HIST_WORDS = LANES * STRIDE + LANES
OUTW = 272

CNT_SHIFT = 20
FRAC_BITS = 9
FRAC_SCALE = float(1 << FRAC_BITS)
PACK_C = float(1 << CNT_SHIFT) + 0.5
FRAC_MASK = (1 << CNT_SHIFT) - 1
INV_FRAC_SCALE = 1.0 / FRAC_SCALE


@functools.cache
def _build_hist_kernel():
    mesh = plsc.VectorSubcoreMesh(core_axis_name="c", subcore_axis_name="s")
    return pl.kernel(
        _hist_body,
        out_type=jax.ShapeDtypeStruct((NW, OUTW), jnp.float32),
        mesh=mesh,
        compiler_params=pltpu.CompilerParams(needs_layout_passes=False),
        scratch_types=[
            pltpu.VMEM((CH,), jnp.float32),
            pltpu.VMEM((CH,), jnp.float32),
            pltpu.VMEM((HIST_WORDS,), jnp.int32),
            pltpu.VMEM((HIST_WORDS,), jnp.float32),
            pltpu.VMEM((HIST_WORDS,), jnp.float32),
            pltpu.VMEM((OUTW + LANES,), jnp.float32),
            pltpu.VMEM((OUTW,), jnp.float32),
            pltpu.SemaphoreType.DMA,
            pltpu.SemaphoreType.DMA,
        ],
    )


def _hist_body(
    img_hbm, out_hbm, buf0, buf1, phist, nacc, facc, frow, outrow, sem0, sem1
):
    wid = lax.axis_index("s") * NC + lax.axis_index("c")
    base = wid * PER_W

    zero = jnp.zeros((LANES,), jnp.float32)
    izero = jnp.zeros((LANES,), jnp.int32)

    def zbody(i, carry):
        phist[pl.ds(i * LANES, LANES)] = izero
        nacc[pl.ds(i * LANES, LANES)] = zero
        facc[pl.ds(i * LANES, LANES)] = zero
        return carry

    lax.fori_loop(0, HIST_WORDS // LANES, zbody, 0)

    lane_base = lax.iota(jnp.int32, LANES) * STRIDE

    ZBIAS = float(1 << 23)
    ZSCALE = INV_DH * FRAC_SCALE
    ZINT = 0x4B000000

    def scan(bref):
        @plsc.parallel_loop(0, CH, LANES, unroll=UNROLL)
        def _(i):
            x = bref[pl.ds(i, LANES)]
            zi = plsc.bitcast(x * ZSCALE + ZBIAS, jnp.int32)
            si = lax.shift_right_logical(zi, FRAC_BITS) + (
                lane_base - (ZINT >> FRAC_BITS)
            )
            pv = (zi & (int(FRAC_SCALE) - 1)) + (1 << CNT_SHIFT)
            plsc.addupdate_scatter(phist, [si], pv)

    def flush():
        @plsc.parallel_loop(0, HIST_WORDS, LANES, unroll=4)
        def _(g):
            s = pl.ds(g, LANES)
            p = phist[s]
            c = lax.shift_right_logical(p, CNT_SHIFT)
            fr = p & FRAC_MASK
            nacc[s] = nacc[s] + c.astype(jnp.float32)
            facc[s] = facc[s] + fr.astype(jnp.float32) * INV_FRAC_SCALE
            phist[s] = izero

    def process(bref):
        scan(bref)
        flush()

    def copy(c, bref, sem):
        return pltpu.make_async_copy(
            img_hbm.at[pl.ds(base + c * CH, CH)], bref, sem
        )

    copy(0, buf0, sem0).start()

    def chunk_pair(p, carry):
        c0 = 2 * p
        copy(c0, buf0, sem0).wait()
        copy(c0 + 1, buf1, sem1).start()
        process(buf0)
        copy(c0 + 1, buf1, sem1).wait()

        @pl.when(p < NCH // 2 - 1)
        def _():
            copy(c0 + 2, buf0, sem0).start()

        process(buf1)
        return carry

    lax.fori_loop(0, NCH // 2, chunk_pair, 0)

    frow[pl.ds(0, LANES)] = zero
    frow[pl.ds(OUTW, LANES)] = zero
    for v in range(OUTW // LANES):
        nrow = nacc[pl.ds(v * LANES, LANES)]
        fr = facc[pl.ds(v * LANES, LANES)]
        for l in range(1, LANES):
            off = l * STRIDE + v * LANES
            nrow = nrow + nacc[pl.ds(off, LANES)]
            fr = fr + facc[pl.ds(off, LANES)]
        outrow[pl.ds(v * LANES, LANES)] = nrow - fr
        frow[pl.ds(v * LANES + 1, LANES)] = fr
    for v in range(OUTW // LANES):
        s = pl.ds(v * LANES, LANES)
        outrow[s] = outrow[s] + frow[s]

    pltpu.sync_copy(outrow, out_hbm.at[wid])


def kernel(img):
    img = img.reshape(-1)
    parts = _build_hist_kernel()(img)
    return jnp.sum(parts[:, :NBIN], axis=0)
